# Initial kernel scaffold; baseline (speedup 1.0000x reference)
import jax
import jax.numpy as jnp
from jax.experimental import pallas as pl


def kernel(norm, feat, viewdir, kn_params):
    raise NotImplementedError("write your pallas kernel here")


# speedup vs baseline: 3.4728x; 3.4728x over previous
import functools

import jax
import jax.numpy as jnp
from jax.experimental import pallas as pl
from jax.experimental.pallas import tpu as pltpu

_N = 8192
_FEAT = 64
_HID = 64
_MAP = 8
_NETS = _MAP * _MAP
_FREQS = 4
_IN1 = _FEAT + 3 + 3 * 2 * _FREQS
_IN1P = 96
_TILE = 512


def _dense_body(norm_ref, feat_ref, view_ref, w1_ref, b1_ref, w2_ref, out_ref):
    feat = feat_ref[...]
    v = view_ref[...]
    norm = norm_ref[...]

    angs = [v * ((2.0 ** k) * jnp.pi) for k in range(_FREQS)]
    ang = jnp.concatenate(angs, axis=1)
    zeros_pad = jnp.zeros((feat.shape[0], _IN1P - _IN1), dtype=feat.dtype)
    x = jnp.concatenate([feat, v, jnp.sin(ang), jnp.cos(ang), zeros_pad],
                        axis=1)

    coords = jnp.clip(jnp.floor(norm[:, :2] * _MAP), 0, _MAP - 1).astype(jnp.int32)
    netid = (coords[:, 0:1] * _MAP + coords[:, 1:2])

    h = jax.lax.dot_general(x, w1_ref[...], (((1,), (0,)), ((), ())),
                            preferred_element_type=jnp.float32)
    h = jax.nn.relu(h + b1_ref[...])
    col_exp = jax.lax.broadcasted_iota(jnp.int32, h.shape, 1) // _HID
    hm = jnp.where(netid == col_exp, h, 0.0)
    out_ref[...] = jax.lax.dot_general(hm, w2_ref[...], (((1,), (0,)), ((), ())),
                                       preferred_element_type=jnp.float32)


def kernel(norm, feat, viewdir, kn_params):
    o0 = _IN1 * _HID
    w1 = kn_params[:, :o0].reshape(_NETS, _IN1, _HID)
    w1 = jnp.transpose(w1, (1, 0, 2)).reshape(_IN1, _NETS * _HID)
    w1 = jnp.pad(w1, ((0, _IN1P - _IN1), (0, 0)))
    b1 = kn_params[:, o0:o0 + _HID].reshape(1, _NETS * _HID)
    w2 = kn_params[:, o0 + _HID:].reshape(_NETS * _HID, 3)

    grid = (_N // _TILE,)
    return pl.pallas_call(
        _dense_body,
        grid=grid,
        in_specs=[
            pl.BlockSpec((_TILE, 3), lambda i: (i, 0)),
            pl.BlockSpec((_TILE, _FEAT), lambda i: (i, 0)),
            pl.BlockSpec((_TILE, 3), lambda i: (i, 0)),
            pl.BlockSpec((_IN1P, _NETS * _HID), lambda i: (0, 0)),
            pl.BlockSpec((1, _NETS * _HID), lambda i: (0, 0)),
            pl.BlockSpec((_NETS * _HID, 3), lambda i: (0, 0)),
        ],
        out_specs=pl.BlockSpec((_TILE, 3), lambda i: (i, 0)),
        out_shape=jax.ShapeDtypeStruct((_N, 3), jnp.float32),
    )(norm, feat, viewdir, w1, b1, w2)
